# baseline (speedup 1.0000x reference)
import functools

import jax
import jax.numpy as jnp
from jax import lax
from jax.experimental import pallas as pl
from jax.experimental.pallas import tpu as pltpu
from jax.experimental.pallas import tpu_sc as plsc

_SAMPLE_NUM = 1152
_EPS = 1e-5

try:
    _info = plsc.get_sparse_core_info()
    _NC, _NS = _info.num_cores, _info.num_subcores
except Exception:
    _NC, _NS = 2, 16
_NW = _NC * _NS


def _sc_gather_add(total0, total1, C):
    r0 = total0 // _NW
    r1 = total1 // _NW
    CH = 48
    n_ch = r1 // CH
    mesh = plsc.VectorSubcoreMesh(core_axis_name="c", subcore_axis_name="s")

    @functools.partial(
        pl.kernel,
        mesh=mesh,
        out_type=(
            jax.ShapeDtypeStruct((total0, C), jnp.float32),
            jax.ShapeDtypeStruct((total1, C), jnp.float32),
        ),
        scratch_types=[
            pltpu.VMEM((r0,), jnp.int32),
            pltpu.VMEM((r0, C), jnp.float32),
            pltpu.VMEM((r0, C), jnp.float32),
            pltpu.VMEM((CH,), jnp.int32),
            pltpu.VMEM((CH,), jnp.int32),
            pltpu.VMEM((CH, C), jnp.float32),
            pltpu.VMEM((CH, C), jnp.float32),
            pltpu.SemaphoreType.DMA,
        ],
    )
    def k(xg_hbm, pg_hbm, xa_hbm, idx_hbm, posd_hbm, pe_hbm,
          out0_hbm, out1_hbm,
          pg_v, g_x, g_pf, idx_v, posd_v, a_x, a_pf, sem):
        wid = lax.axis_index("s") * _NC + lax.axis_index("c")

        b0 = wid * r0
        pltpu.sync_copy(pg_hbm.at[pl.ds(b0, r0)], pg_v)
        cx = pltpu.async_copy(xg_hbm.at[pl.ds(b0, r0)], g_x, sem)
        cp = pltpu.async_copy(pe_hbm.at[pg_v], g_pf, sem)
        cx.wait()
        cp.wait()

        def add_row0(r, carry):
            for c in range(0, C, 16):
                g_x[r, pl.ds(c, 16)] = (g_x[r, pl.ds(c, 16)]
                                        + g_pf[r, pl.ds(c, 16)])
            return carry

        lax.fori_loop(0, r0, add_row0, 0)
        pltpu.sync_copy(g_x, out0_hbm.at[pl.ds(b0, r0)])

        def add_row1(r, carry):
            for c in range(0, C, 16):
                a_x[r, pl.ds(c, 16)] = (a_x[r, pl.ds(c, 16)]
                                        + a_pf[r, pl.ds(c, 16)])
            return carry

        for k in range(n_ch):
            b1 = wid * r1 + k * CH
            pltpu.sync_copy(idx_hbm.at[pl.ds(b1, CH)], idx_v)
            pltpu.sync_copy(posd_hbm.at[pl.ds(b1, CH)], posd_v)
            ca = pltpu.async_copy(xa_hbm.at[idx_v], a_x, sem)
            cb = pltpu.async_copy(pe_hbm.at[posd_v], a_pf, sem)
            ca.wait()
            cb.wait()
            lax.fori_loop(0, CH, add_row1, 0)
            pltpu.sync_copy(a_x, out1_hbm.at[pl.ds(b1, CH)])

    return k


def _rank_kernel(N, S):
    JB = N // 128

    def body(col_ref, row_ref, rank_ref):
        s_row = row_ref[0, 0:1, :]
        ilt_row = lax.broadcasted_iota(jnp.int32, (1, N), 1).astype(
            jnp.float32)

        def jblock(jb, acc):
            s_col = col_ref[0, pl.ds(jb * 128, 128), :]
            gt = (s_col > s_row).astype(jnp.float32)
            return acc + jnp.sum(gt, axis=0, keepdims=True)

        cnt = lax.fori_loop(0, JB, jblock, jnp.zeros((1, N), jnp.float32))
        total = jnp.sum(cnt)

        def tie_fix(cnt0):
            def jblock2(jb, acc):
                s_col = col_ref[0, pl.ds(jb * 128, 128), :]
                jidx = (lax.broadcasted_iota(jnp.int32, (128, 1), 0)
                        .astype(jnp.float32)
                        + jnp.float32(128) * jb.astype(jnp.float32))
                eqlt = ((s_col == s_row) & (jidx < ilt_row))
                return acc + jnp.sum(eqlt.astype(jnp.float32), axis=0,
                                     keepdims=True)
            return lax.fori_loop(0, JB, jblock2, cnt0)

        expected = jnp.float32(N * (N - 1) // 2)
        cnt = lax.cond(total != expected, tie_fix, lambda c: c, cnt)
        rank_ref[0, 0:1, :] = cnt.astype(jnp.int32)

    return pl.pallas_call(
        body,
        grid=(4,),
        in_specs=[
            pl.BlockSpec((1, N, 1), lambda b: (b, 0, 0)),
            pl.BlockSpec((1, 1, N), lambda b: (b, 0, 0)),
        ],
        out_specs=pl.BlockSpec((1, 1, N), lambda b: (b, 0, 0)),
        out_shape=jax.ShapeDtypeStruct((4, 1, N), jnp.int32),
    )


def kernel(x_grid, x_ada, pos_grid, pos_ada, pos_embed, norm_w, norm_b,
           conf_w, conf_b):
    B, N_g, C = x_grid.shape
    N = x_ada.shape[1]
    S = _SAMPLE_NUM

    mu = jnp.mean(x_ada, axis=-1, keepdims=True)
    var = jnp.var(x_ada, axis=-1, keepdims=True)
    normed = (x_ada - mu) / jnp.sqrt(var + _EPS) * norm_w + norm_b
    conf = normed @ conf_w + conf_b
    conf = jax.nn.softmax(conf, axis=1) * N

    s = conf[..., 0]
    rank = _rank_kernel(N, S)(conf, s[:, None, :])[:, 0, :]

    tok = jnp.broadcast_to(jnp.arange(N, dtype=jnp.int32), (B, N))
    idx = jnp.zeros((B, S), jnp.int32).at[
        jnp.arange(B, dtype=jnp.int32)[:, None], rank].set(
            tok, mode="drop")
    pos_down = jnp.take_along_axis(pos_ada, idx, axis=1)

    idx_g = (idx + jnp.arange(B, dtype=idx.dtype)[:, None] * N).reshape(-1)
    posd_f = pos_down.reshape(-1).astype(jnp.int32)
    pg_f = pos_grid.reshape(-1).astype(jnp.int32)
    pe = pos_embed[0]

    sc = _sc_gather_add(B * N_g, B * S, C)
    out0_f, out1_f = sc(
        x_grid.reshape(B * N_g, C), pg_f,
        x_ada.reshape(B * N, C), idx_g.astype(jnp.int32), posd_f, pe)
    out0 = out0_f.reshape(B, N_g, C)
    out1 = out1_f.reshape(B, S, C)
    return out0, out1, pos_grid, pos_ada


# speedup vs baseline: 1.3398x; 1.3398x over previous
import functools

import jax
import jax.numpy as jnp
from jax import lax
from jax.experimental import pallas as pl
from jax.experimental.pallas import tpu as pltpu
from jax.experimental.pallas import tpu_sc as plsc

_SAMPLE_NUM = 1152
_EPS = 1e-5

try:
    _info = plsc.get_sparse_core_info()
    _NC, _NS = _info.num_cores, _info.num_subcores
except Exception:
    _NC, _NS = 2, 16
_NW = _NC * _NS


def _sc_gather_add(total0, total1, C):
    r0 = total0 // _NW
    r1 = total1 // _NW
    CH = 48
    n_ch = r1 // CH
    mesh = plsc.VectorSubcoreMesh(core_axis_name="c", subcore_axis_name="s")

    @functools.partial(
        pl.kernel,
        mesh=mesh,
        out_type=(
            jax.ShapeDtypeStruct((total0, C), jnp.float32),
            jax.ShapeDtypeStruct((total1, C), jnp.float32),
        ),
        scratch_types=[
            pltpu.VMEM((r0,), jnp.int32),
            pltpu.VMEM((r0, C), jnp.float32),
            pltpu.VMEM((r0, C), jnp.float32),
            pltpu.VMEM((CH,), jnp.int32),
            pltpu.VMEM((CH,), jnp.int32),
            pltpu.VMEM((CH, C), jnp.float32),
            pltpu.VMEM((CH, C), jnp.float32),
            pltpu.SemaphoreType.DMA,
        ],
    )
    def k(xg_hbm, pg_hbm, xa_hbm, idx_hbm, posd_hbm, pe_hbm,
          out0_hbm, out1_hbm,
          pg_v, g_x, g_pf, idx_v, posd_v, a_x, a_pf, sem):
        wid = lax.axis_index("s") * _NC + lax.axis_index("c")

        b0 = wid * r0
        pltpu.sync_copy(pg_hbm.at[pl.ds(b0, r0)], pg_v)
        cx = pltpu.async_copy(xg_hbm.at[pl.ds(b0, r0)], g_x, sem)
        cp = pltpu.async_copy(pe_hbm.at[pg_v], g_pf, sem)
        cx.wait()
        cp.wait()

        def add_row0(r, carry):
            for c in range(0, C, 16):
                g_x[r, pl.ds(c, 16)] = (g_x[r, pl.ds(c, 16)]
                                        + g_pf[r, pl.ds(c, 16)])
            return carry

        lax.fori_loop(0, r0, add_row0, 0)
        pltpu.sync_copy(g_x, out0_hbm.at[pl.ds(b0, r0)])

        def add_row1(r, carry):
            for c in range(0, C, 16):
                a_x[r, pl.ds(c, 16)] = (a_x[r, pl.ds(c, 16)]
                                        + a_pf[r, pl.ds(c, 16)])
            return carry

        for k in range(n_ch):
            b1 = wid * r1 + k * CH
            pltpu.sync_copy(idx_hbm.at[pl.ds(b1, CH)], idx_v)
            pltpu.sync_copy(posd_hbm.at[pl.ds(b1, CH)], posd_v)
            ca = pltpu.async_copy(xa_hbm.at[idx_v], a_x, sem)
            cb = pltpu.async_copy(pe_hbm.at[posd_v], a_pf, sem)
            ca.wait()
            cb.wait()
            lax.fori_loop(0, CH, add_row1, 0)
            pltpu.sync_copy(a_x, out1_hbm.at[pl.ds(b1, CH)])

    return k


def _topk_sort_kernel(B, M):
    R = M // 128
    L = M.bit_length() - 1

    def body(key_ref, idx_out_ref):
        v = key_ref[0]
        lane = lax.broadcasted_iota(jnp.int32, (R, 128), 1)
        rowi = lax.broadcasted_iota(jnp.int32, (R, 128), 0)
        ix = rowi * 128 + lane

        for k in range(1, L + 1):
            size = 1 << k
            if size < 128:
                desc = (lane & size) == 0
            elif size < M:
                desc = (rowi & (size // 128)) == 0
            else:
                desc = jnp.full((R, 128), True)
            for j in range(k - 1, -1, -1):
                d = 1 << j
                if d < 128:
                    left = (lane & d) == 0
                    pv = jnp.where(left, jnp.roll(v, -d, axis=1),
                                   jnp.roll(v, d, axis=1))
                    pi = jnp.where(left, jnp.roll(ix, -d, axis=1),
                                   jnp.roll(ix, d, axis=1))
                else:
                    dr = d // 128
                    left = (rowi & dr) == 0
                    pv = jnp.where(left, jnp.roll(v, -dr, axis=0),
                                   jnp.roll(v, dr, axis=0))
                    pi = jnp.where(left, jnp.roll(ix, -dr, axis=0),
                                   jnp.roll(ix, dr, axis=0))
                win = (v > pv) | ((v == pv) & (ix < pi))
                m = (left == desc) == win
                v = jnp.where(m, v, pv)
                ix = jnp.where(m, ix, pi)
        idx_out_ref[0] = ix

    return pl.pallas_call(
        body,
        grid=(B,),
        in_specs=[pl.BlockSpec((1, R, 128), lambda b: (b, 0, 0))],
        out_specs=pl.BlockSpec((1, R, 128), lambda b: (b, 0, 0)),
        out_shape=jax.ShapeDtypeStruct((B, R, 128), jnp.int32),
    )


def kernel(x_grid, x_ada, pos_grid, pos_ada, pos_embed, norm_w, norm_b,
           conf_w, conf_b):
    B, N_g, C = x_grid.shape
    N = x_ada.shape[1]
    S = _SAMPLE_NUM

    mu = jnp.mean(x_ada, axis=-1, keepdims=True)
    var = jnp.var(x_ada, axis=-1, keepdims=True)
    normed = (x_ada - mu) / jnp.sqrt(var + _EPS) * norm_w + norm_b
    conf = normed @ conf_w + conf_b
    conf = jax.nn.softmax(conf, axis=1) * N

    s = conf[..., 0]
    M = 4096
    s_pad = jnp.concatenate(
        [s.reshape(B, N // 128, 128),
         jnp.full((B, (M - N) // 128, 128), -1.0, jnp.float32)], axis=1)
    ix_sorted = _topk_sort_kernel(B, M)(s_pad)
    idx = ix_sorted.reshape(B, M)[:, :S]
    pos_down = jnp.take_along_axis(pos_ada, idx, axis=1)

    idx_g = (idx + jnp.arange(B, dtype=idx.dtype)[:, None] * N).reshape(-1)
    posd_f = pos_down.reshape(-1).astype(jnp.int32)
    pg_f = pos_grid.reshape(-1).astype(jnp.int32)
    pe = pos_embed[0]

    sc = _sc_gather_add(B * N_g, B * S, C)
    out0_f, out1_f = sc(
        x_grid.reshape(B * N_g, C), pg_f,
        x_ada.reshape(B * N, C), idx_g.astype(jnp.int32), posd_f, pe)
    out0 = out0_f.reshape(B, N_g, C)
    out1 = out1_f.reshape(B, S, C)
    return out0, out1, pos_grid, pos_ada


# speedup vs baseline: 1.5488x; 1.1560x over previous
import functools

import jax
import jax.numpy as jnp
from jax import lax
from jax.experimental import pallas as pl
from jax.experimental.pallas import tpu as pltpu
from jax.experimental.pallas import tpu_sc as plsc

_SAMPLE_NUM = 1152
_EPS = 1e-5

try:
    _info = plsc.get_sparse_core_info()
    _NC, _NS = _info.num_cores, _info.num_subcores
except Exception:
    _NC, _NS = 2, 16
_NW = _NC * _NS


def _sc_gather_add(total0, total1, C):
    r0 = total0 // _NW
    r1 = total1 // _NW
    CH = 48
    n_ch = r1 // CH
    mesh = plsc.VectorSubcoreMesh(core_axis_name="c", subcore_axis_name="s")

    @functools.partial(
        pl.kernel,
        mesh=mesh,
        out_type=(
            jax.ShapeDtypeStruct((total0, C), jnp.float32),
            jax.ShapeDtypeStruct((total1, C), jnp.float32),
        ),
        scratch_types=[
            pltpu.VMEM((r0,), jnp.int32),
            pltpu.VMEM((r0, C), jnp.float32),
            pltpu.VMEM((r0, C), jnp.float32),
            pltpu.VMEM((CH,), jnp.int32),
            pltpu.VMEM((CH,), jnp.int32),
            pltpu.VMEM((CH, C), jnp.float32),
            pltpu.VMEM((CH, C), jnp.float32),
            pltpu.SemaphoreType.DMA,
        ],
    )
    def k(xg_hbm, pg_hbm, xa_hbm, idx_hbm, posd_hbm, pe_hbm,
          out0_hbm, out1_hbm,
          pg_v, g_x, g_pf, idx_v, posd_v, a_x, a_pf, sem):
        wid = lax.axis_index("s") * _NC + lax.axis_index("c")

        b0 = wid * r0
        pltpu.sync_copy(pg_hbm.at[pl.ds(b0, r0)], pg_v)
        cx = pltpu.async_copy(xg_hbm.at[pl.ds(b0, r0)], g_x, sem)
        cp = pltpu.async_copy(pe_hbm.at[pg_v], g_pf, sem)
        cx.wait()
        cp.wait()

        def add_row0(r, carry):
            for c in range(0, C, 16):
                g_x[r, pl.ds(c, 16)] = (g_x[r, pl.ds(c, 16)]
                                        + g_pf[r, pl.ds(c, 16)])
            return carry

        lax.fori_loop(0, r0, add_row0, 0)
        pltpu.sync_copy(g_x, out0_hbm.at[pl.ds(b0, r0)])

        def add_row1(r, carry):
            for c in range(0, C, 16):
                a_x[r, pl.ds(c, 16)] = (a_x[r, pl.ds(c, 16)]
                                        + a_pf[r, pl.ds(c, 16)])
            return carry

        for k in range(n_ch):
            b1 = wid * r1 + k * CH
            pltpu.sync_copy(idx_hbm.at[pl.ds(b1, CH)], idx_v)
            pltpu.sync_copy(posd_hbm.at[pl.ds(b1, CH)], posd_v)
            ca = pltpu.async_copy(xa_hbm.at[idx_v], a_x, sem)
            cb = pltpu.async_copy(pe_hbm.at[posd_v], a_pf, sem)
            ca.wait()
            cb.wait()
            lax.fori_loop(0, CH, add_row1, 0)
            pltpu.sync_copy(a_x, out1_hbm.at[pl.ds(b1, CH)])

    return k


def _topk_sort_kernel(B, M):
    R = M // 128
    L = M.bit_length() - 1

    def body(key_ref, idx_out_ref):
        v = key_ref[...]
        lane = lax.broadcasted_iota(jnp.int32, (B, R, 128), 2)
        rowi = lax.broadcasted_iota(jnp.int32, (B, R, 128), 1)
        ix = rowi * 128 + lane

        for k in range(1, L + 1):
            size = 1 << k
            if size < 128:
                desc = (lane & size) == 0
            elif size < M:
                desc = (rowi & (size // 128)) == 0
            else:
                desc = jnp.full((B, R, 128), True)
            for j in range(k - 1, -1, -1):
                d = 1 << j
                if d < 128:
                    left = (lane & d) == 0
                    pv = jnp.where(left, jnp.roll(v, -d, axis=2),
                                   jnp.roll(v, d, axis=2))
                    pi = jnp.where(left, jnp.roll(ix, -d, axis=2),
                                   jnp.roll(ix, d, axis=2))
                else:
                    dr = d // 128
                    left = (rowi & dr) == 0
                    pv = jnp.where(left, jnp.roll(v, -dr, axis=1),
                                   jnp.roll(v, dr, axis=1))
                    pi = jnp.where(left, jnp.roll(ix, -dr, axis=1),
                                   jnp.roll(ix, dr, axis=1))
                win = (v > pv) | ((v == pv) & (ix < pi))
                m = (left == desc) == win
                v = jnp.where(m, v, pv)
                ix = jnp.where(m, ix, pi)
        idx_out_ref[...] = ix

    return pl.pallas_call(
        body,
        out_shape=jax.ShapeDtypeStruct((B, R, 128), jnp.int32),
    )


def kernel(x_grid, x_ada, pos_grid, pos_ada, pos_embed, norm_w, norm_b,
           conf_w, conf_b):
    B, N_g, C = x_grid.shape
    N = x_ada.shape[1]
    S = _SAMPLE_NUM

    mu = jnp.mean(x_ada, axis=-1, keepdims=True)
    var = jnp.var(x_ada, axis=-1, keepdims=True)
    normed = (x_ada - mu) / jnp.sqrt(var + _EPS) * norm_w + norm_b
    conf = normed @ conf_w + conf_b
    conf = jax.nn.softmax(conf, axis=1) * N

    s = conf[..., 0]
    M = 4096
    s_pad = jnp.concatenate(
        [s.reshape(B, N // 128, 128),
         jnp.full((B, (M - N) // 128, 128), -1.0, jnp.float32)], axis=1)
    ix_sorted = _topk_sort_kernel(B, M)(s_pad)
    idx = ix_sorted.reshape(B, M)[:, :S]
    pos_down = jnp.take_along_axis(pos_ada, idx, axis=1)

    idx_g = (idx + jnp.arange(B, dtype=idx.dtype)[:, None] * N).reshape(-1)
    posd_f = pos_down.reshape(-1).astype(jnp.int32)
    pg_f = pos_grid.reshape(-1).astype(jnp.int32)
    pe = pos_embed[0]

    sc = _sc_gather_add(B * N_g, B * S, C)
    out0_f, out1_f = sc(
        x_grid.reshape(B * N_g, C), pg_f,
        x_ada.reshape(B * N, C), idx_g.astype(jnp.int32), posd_f, pe)
    out0 = out0_f.reshape(B, N_g, C)
    out1 = out1_f.reshape(B, S, C)
    return out0, out1, pos_grid, pos_ada


# speedup vs baseline: 1.7909x; 1.1563x over previous
import functools

import jax
import jax.numpy as jnp
from jax import lax
from jax.experimental import pallas as pl
from jax.experimental.pallas import tpu as pltpu
from jax.experimental.pallas import tpu_sc as plsc

_SAMPLE_NUM = 1152
_EPS = 1e-5

try:
    _info = plsc.get_sparse_core_info()
    _NC, _NS = _info.num_cores, _info.num_subcores
except Exception:
    _NC, _NS = 2, 16
_NW = _NC * _NS


def _sc_gather_add(B, Ng, N, S, C, M):
    r0 = B * Ng // _NW
    r1 = B * S // _NW
    CH = 24
    n0 = r0 // CH
    n1 = r1 // CH
    WB = _NW // B
    mesh = plsc.VectorSubcoreMesh(core_axis_name="c", subcore_axis_name="s")

    @functools.partial(
        pl.kernel,
        mesh=mesh,
        out_type=(
            jax.ShapeDtypeStruct((B * Ng, C), jnp.float32),
            jax.ShapeDtypeStruct((B * S, C), jnp.float32),
        ),
        scratch_types=[
            pltpu.VMEM((r0,), jnp.int32),
            pltpu.VMEM((r1,), jnp.int32),
            pltpu.VMEM((r1,), jnp.int32),
            pltpu.VMEM((CH, C), jnp.float32),
            pltpu.VMEM((CH, C), jnp.float32),
            pltpu.VMEM((CH, C), jnp.float32),
            pltpu.VMEM((CH, C), jnp.float32),
            pltpu.VMEM((CH, C), jnp.float32),
            pltpu.VMEM((CH, C), jnp.float32),
        ] + [pltpu.SemaphoreType.DMA] * 6,
        name="sc_gather_add",
    )
    def k(xg_hbm, pg_hbm, xa_hbm, idx_hbm, pos_hbm, pe_hbm,
          out0_hbm, out1_hbm,
          pg_v, idx_v, posd_v, g_x, g_pf, ax0, apf0, ax1, apf1,
          semA, semC, semD, semE0, semE1, semG):
        wid = lax.axis_index("s") * _NC + lax.axis_index("c")
        b0 = wid * r0
        boff = (wid // WB) * M + (wid % WB) * r1
        obase = wid * r1

        def add_rows(xbuf, pfbuf):
            def add_row(r, carry):
                for c in range(0, C, 16):
                    xbuf[r, pl.ds(c, 16)] = (xbuf[r, pl.ds(c, 16)]
                                             + pfbuf[r, pl.ds(c, 16)])
                return carry
            lax.fori_loop(0, CH, add_row, 0)

        pltpu.sync_copy(idx_hbm.at[pl.ds(boff, r1)], idx_v)
        c_pd = pltpu.async_copy(pos_hbm.at[idx_v], posd_v, semD)
        pltpu.sync_copy(pg_hbm.at[pl.ds(b0, r0)], pg_v)
        c_pd.wait()

        abufs = ((ax0, apf0, semE0), (ax1, apf1, semE1))
        pend = [None, None]
        stores = []

        def start1(kc):
            xb, pb, sem = abufs[kc % 2]
            cx = pltpu.async_copy(
                xa_hbm.at[idx_v.at[pl.ds(kc * CH, CH)]], xb, sem)
            cp = pltpu.async_copy(
                pe_hbm.at[posd_v.at[pl.ds(kc * CH, CH)]], pb, sem)
            pend[kc % 2] = (cx, cp)

        start1(0)
        start1(1)
        for kc in range(n1):
            xb, pb, _ = abufs[kc % 2]
            cx, cp = pend[kc % 2]
            cx.wait()
            cp.wait()
            add_rows(xb, pb)
            st = pltpu.async_copy(
                xb, out1_hbm.at[pl.ds(obase + kc * CH, CH)], semG)
            stores.append(st)
            if kc + 2 < n1:
                st.wait()
                start1(kc + 2)

        for kc in range(n0):
            base = b0 + kc * CH
            cg = pltpu.async_copy(xg_hbm.at[pl.ds(base, CH)], g_x, semA)
            cf = pltpu.async_copy(
                pe_hbm.at[pg_v.at[pl.ds(kc * CH, CH)]], g_pf, semA)
            cg.wait()
            cf.wait()
            add_rows(g_x, g_pf)
            pltpu.sync_copy(g_x, out0_hbm.at[pl.ds(base, CH)])

        for st in stores[-2:]:
            st.wait()

    return k


def _topk_sort_kernel(B, M):
    R = M // 128
    L = M.bit_length() - 1

    def body(key_ref, idx_out_ref):
        v = key_ref[...]
        lane = lax.broadcasted_iota(jnp.int32, (B, R, 128), 2)
        rowi = lax.broadcasted_iota(jnp.int32, (B, R, 128), 1)
        bi = lax.broadcasted_iota(jnp.int32, (B, R, 128), 0)
        ix = bi * 2304 + rowi * 128 + lane

        for k in range(1, L + 1):
            size = 1 << k
            if size < 128:
                desc = (lane & size) == 0
            elif size < M:
                desc = (rowi & (size // 128)) == 0
            else:
                desc = jnp.full((B, R, 128), True)
            for j in range(k - 1, -1, -1):
                d = 1 << j
                if d < 128:
                    left = (lane & d) == 0
                    pv = jnp.where(left, jnp.roll(v, -d, axis=2),
                                   jnp.roll(v, d, axis=2))
                    pi = jnp.where(left, jnp.roll(ix, -d, axis=2),
                                   jnp.roll(ix, d, axis=2))
                else:
                    dr = d // 128
                    left = (rowi & dr) == 0
                    pv = jnp.where(left, jnp.roll(v, -dr, axis=1),
                                   jnp.roll(v, dr, axis=1))
                    pi = jnp.where(left, jnp.roll(ix, -dr, axis=1),
                                   jnp.roll(ix, dr, axis=1))
                win = (v > pv) | ((v == pv) & (ix < pi))
                m = (left == desc) == win
                v = jnp.where(m, v, pv)
                ix = jnp.where(m, ix, pi)
        idx_out_ref[...] = ix

    return pl.pallas_call(
        body,
        out_shape=jax.ShapeDtypeStruct((B, R, 128), jnp.int32),
    )


def kernel(x_grid, x_ada, pos_grid, pos_ada, pos_embed, norm_w, norm_b,
           conf_w, conf_b):
    B, N_g, C = x_grid.shape
    N = x_ada.shape[1]
    S = _SAMPLE_NUM

    mu = jnp.mean(x_ada, axis=-1, keepdims=True)
    var = jnp.var(x_ada, axis=-1, keepdims=True)
    normed = (x_ada - mu) / jnp.sqrt(var + _EPS) * norm_w + norm_b
    conf = normed @ conf_w + conf_b
    conf = jax.nn.softmax(conf, axis=1) * N

    s = conf[..., 0]
    M = 4096
    s_pad = jnp.concatenate(
        [s.reshape(B, N // 128, 128),
         jnp.full((B, (M - N) // 128, 128), -1.0, jnp.float32)], axis=1)
    ix_sorted = _topk_sort_kernel(B, M)(s_pad)

    pg_f = pos_grid.reshape(-1).astype(jnp.int32)
    pe = pos_embed[0]

    sc = _sc_gather_add(B, N_g, N, S, C, M)
    out0_f, out1_f = sc(
        x_grid.reshape(B * N_g, C), pg_f,
        x_ada.reshape(B * N, C), ix_sorted.reshape(B * M),
        pos_ada.reshape(-1).astype(jnp.int32), pe)
    out0 = out0_f.reshape(B, N_g, C)
    out1 = out1_f.reshape(B, S, C)
    return out0, out1, pos_grid, pos_ada
